# baseline (speedup 1.0000x reference)
import functools

import jax
import jax.numpy as jnp
from jax import lax
from jax.experimental import pallas as pl
from jax.experimental.pallas import tpu as pltpu
from jax.experimental.pallas import tpu_sc as plsc

_LANES = 16
_NBUF = 2


@functools.lru_cache(maxsize=None)
def _make_sc_pool(n_seg: int, seg_rows: int, units: int, col0: int,
                  chunk_rows: int):
    n_cores = 2
    sc_cols = units - col0
    w_cols = sc_cols
    seg_per_core = n_seg // n_cores
    n_grp = w_cols // _LANES
    n_chunks = seg_rows // chunk_rows
    assert n_chunks % _NBUF == 0 and chunk_rows % 8 == 0
    mesh = plsc.VectorSubcoreMesh(core_axis_name="c", subcore_axis_name="s")

    @functools.partial(
        pl.kernel,
        mesh=mesh,
        out_type=jax.ShapeDtypeStruct((n_seg, sc_cols), jnp.float32),
        scratch_types=(
            [pltpu.VMEM((chunk_rows, w_cols), jnp.float32)] * _NBUF
            + [
                pltpu.VMEM((_LANES,), jnp.float32),
                pltpu.VMEM((w_cols,), jnp.float32),
            ]
            + [pltpu.SemaphoreType.DMA] * _NBUF
        ),
    )
    def sc_pool(x_hbm, sz_hbm, out_hbm, *refs):
        bufs = refs[:_NBUF]
        szv, outv = refs[_NBUF:_NBUF + 2]
        sems = refs[_NBUF + 2:]
        core = lax.axis_index("c")
        sub = lax.axis_index("s")
        seg = core * seg_per_core + sub

        @pl.when(sub < seg_per_core)
        def _():
            row0 = seg * seg_rows

            def src_slice(ci):
                off = pl.multiple_of(row0 + ci * chunk_rows, 8)
                return x_hbm.at[pl.ds(off, chunk_rows), pl.ds(col0, w_cols)]

            for b in range(_NBUF):
                pltpu.async_copy(src_slice(b), bufs[b], sems[b])
            acc0 = tuple(
                jnp.zeros((_LANES,), jnp.float32) for _ in range(n_grp))

            def outer(g, acc):
                for b in range(_NBUF):
                    ci = g * _NBUF + b
                    pltpu.make_async_copy(
                        src_slice(ci), bufs[b], sems[b]).wait()

                    def body(r, carry, b=b):
                        return tuple(
                            carry[gr] + bufs[b][r, pl.ds(gr * _LANES, _LANES)]
                            for gr in range(n_grp)
                        )

                    acc = lax.fori_loop(0, chunk_rows, body, acc)

                    @pl.when(ci + _NBUF < n_chunks)
                    def _(b=b, ci=ci):
                        pltpu.async_copy(
                            src_slice(ci + _NBUF), bufs[b], sems[b])
                return acc

            acc = lax.fori_loop(0, n_chunks // _NBUF, outer, acc0)

            pltpu.sync_copy(sz_hbm.at[seg], szv)
            s = szv[...]
            for g in range(n_grp):
                outv[pl.ds(g * _LANES, _LANES)] = acc[g] / s
            pltpu.sync_copy(outv, out_hbm.at[seg])

    return sc_pool


@functools.lru_cache(maxsize=None)
def _make_tc_pool(n_seg: int, seg_rows: int, units: int, tc_cols: int,
                  blk_rows: int):
    n_blk = seg_rows // blk_rows

    def body(x_ref, r_ref, o_ref):
        @pl.when(pl.program_id(0) == 0)
        def _():
            o_ref[...] = jnp.zeros_like(o_ref)
        o_ref[...] += jnp.sum(x_ref[...], axis=1)

        @pl.when(pl.program_id(0) == n_blk - 1)
        def _():
            o_ref[...] = o_ref[...] * r_ref[...]

    return pl.pallas_call(
        body,
        grid=(n_blk,),
        in_specs=[
            pl.BlockSpec((n_seg, blk_rows, tc_cols), lambda k: (0, k, 0)),
            pl.BlockSpec((n_seg, 1), lambda k: (0, 0)),
        ],
        out_specs=pl.BlockSpec((n_seg, tc_cols), lambda k: (0, 0)),
        out_shape=jax.ShapeDtypeStruct((n_seg, tc_cols), jnp.float32),
    )


def kernel(x, nclasses, nfeature):
    n_seg = nclasses.shape[0]
    units = x.shape[1]
    seg_rows = x.shape[0] // n_seg
    tc_cols = 384
    chunk_rows = 400
    blk_rows = 400

    sizes = (nclasses * nfeature).astype(jnp.float32)
    recip = (1.0 / sizes)[:, None]
    sz_b = jnp.broadcast_to(sizes[:, None], (n_seg, _LANES))
    x3 = jnp.reshape(x, (n_seg, seg_rows, units))

    sc_out = _make_sc_pool(n_seg, seg_rows, units, tc_cols, chunk_rows)(x, sz_b)
    tc_out = _make_tc_pool(n_seg, seg_rows, units, tc_cols, blk_rows)(x3, recip)
    return jnp.concatenate([tc_out, sc_out], axis=1)


# speedup vs baseline: 1.0202x; 1.0202x over previous
import functools

import jax
import jax.numpy as jnp
from jax import lax
from jax.experimental import pallas as pl
from jax.experimental.pallas import tpu as pltpu
from jax.experimental.pallas import tpu_sc as plsc

_LANES = 16
_NBUF = 2


@functools.lru_cache(maxsize=None)
def _make_sc_pool(n_seg: int, seg_rows: int, units: int, col0: int,
                  chunk_rows: int):
    n_cores = 1
    sc_cols = units - col0
    w_cols = sc_cols
    seg_per_core = n_seg // n_cores
    n_grp = w_cols // _LANES
    n_chunks = seg_rows // chunk_rows
    assert n_chunks % _NBUF == 0 and chunk_rows % 8 == 0
    mesh = plsc.VectorSubcoreMesh(core_axis_name="c", subcore_axis_name="s",
                                  num_cores=n_cores)

    @functools.partial(
        pl.kernel,
        mesh=mesh,
        out_type=jax.ShapeDtypeStruct((n_seg, sc_cols), jnp.float32),
        scratch_types=(
            [pltpu.VMEM((chunk_rows, w_cols), jnp.float32)] * _NBUF
            + [
                pltpu.VMEM((_LANES,), jnp.float32),
                pltpu.VMEM((w_cols,), jnp.float32),
            ]
            + [pltpu.SemaphoreType.DMA] * _NBUF
        ),
    )
    def sc_pool(x_hbm, sz_hbm, out_hbm, *refs):
        bufs = refs[:_NBUF]
        szv, outv = refs[_NBUF:_NBUF + 2]
        sems = refs[_NBUF + 2:]
        core = lax.axis_index("c")
        sub = lax.axis_index("s")
        seg = core * seg_per_core + sub

        @pl.when(sub < seg_per_core)
        def _():
            row0 = seg * seg_rows

            def src_slice(ci):
                off = pl.multiple_of(row0 + ci * chunk_rows, 8)
                return x_hbm.at[pl.ds(off, chunk_rows), pl.ds(col0, w_cols)]

            for b in range(_NBUF):
                pltpu.async_copy(src_slice(b), bufs[b], sems[b])
            acc0 = tuple(
                jnp.zeros((_LANES,), jnp.float32) for _ in range(n_grp))

            def outer(g, acc):
                for b in range(_NBUF):
                    ci = g * _NBUF + b
                    pltpu.make_async_copy(
                        src_slice(ci), bufs[b], sems[b]).wait()

                    def body(r, carry, b=b):
                        return tuple(
                            carry[gr] + bufs[b][r, pl.ds(gr * _LANES, _LANES)]
                            for gr in range(n_grp)
                        )

                    acc = lax.fori_loop(0, chunk_rows, body, acc)

                    @pl.when(ci + _NBUF < n_chunks)
                    def _(b=b, ci=ci):
                        pltpu.async_copy(
                            src_slice(ci + _NBUF), bufs[b], sems[b])
                return acc

            acc = lax.fori_loop(0, n_chunks // _NBUF, outer, acc0)

            pltpu.sync_copy(sz_hbm.at[seg], szv)
            s = szv[...]
            for g in range(n_grp):
                outv[pl.ds(g * _LANES, _LANES)] = acc[g] / s
            pltpu.sync_copy(outv, out_hbm.at[seg])

    return sc_pool


@functools.lru_cache(maxsize=None)
def _make_tc_pool(n_seg: int, seg_rows: int, units: int, tc_cols: int,
                  blk_rows: int):
    n_blk = seg_rows // blk_rows

    def body(x_ref, r_ref, o_ref):
        @pl.when(pl.program_id(0) == 0)
        def _():
            o_ref[...] = jnp.zeros_like(o_ref)
        o_ref[...] += jnp.sum(x_ref[...], axis=1)

        @pl.when(pl.program_id(0) == n_blk - 1)
        def _():
            o_ref[...] = o_ref[...] * r_ref[...]

    return pl.pallas_call(
        body,
        grid=(n_blk,),
        in_specs=[
            pl.BlockSpec((n_seg, blk_rows, tc_cols), lambda k: (0, k, 0)),
            pl.BlockSpec((n_seg, 1), lambda k: (0, 0)),
        ],
        out_specs=pl.BlockSpec((n_seg, tc_cols), lambda k: (0, 0)),
        out_shape=jax.ShapeDtypeStruct((n_seg, tc_cols), jnp.float32),
    )


def kernel(x, nclasses, nfeature):
    n_seg = nclasses.shape[0]
    units = x.shape[1]
    seg_rows = x.shape[0] // n_seg
    tc_cols = 384
    chunk_rows = 400
    blk_rows = 400

    sizes = (nclasses * nfeature).astype(jnp.float32)
    recip = (1.0 / sizes)[:, None]
    sz_b = jnp.broadcast_to(sizes[:, None], (n_seg, _LANES))
    x3 = jnp.reshape(x, (n_seg, seg_rows, units))

    sc_out = _make_sc_pool(n_seg, seg_rows, units, tc_cols, chunk_rows)(x, sz_b)
    tc_out = _make_tc_pool(n_seg, seg_rows, units, tc_cols, blk_rows)(x3, recip)
    return jnp.concatenate([tc_out, sc_out], axis=1)


# speedup vs baseline: 1.0308x; 1.0104x over previous
import functools

import jax
import jax.numpy as jnp
from jax import lax
from jax.experimental import pallas as pl
from jax.experimental.pallas import tpu as pltpu
from jax.experimental.pallas import tpu_sc as plsc

_LANES = 16
_NBUF = 2


@functools.lru_cache(maxsize=None)
def _make_sc_pool(n_seg: int, seg_rows: int, units: int, col0: int,
                  chunk_rows: int):
    n_cores = 1
    sc_cols = units - col0
    w_cols = sc_cols
    seg_per_core = n_seg // n_cores
    n_grp = w_cols // _LANES
    n_chunks = seg_rows // chunk_rows
    assert n_chunks % _NBUF == 0 and chunk_rows % 8 == 0
    mesh = plsc.VectorSubcoreMesh(core_axis_name="c", subcore_axis_name="s",
                                  num_cores=n_cores)

    @functools.partial(
        pl.kernel,
        mesh=mesh,
        out_type=jax.ShapeDtypeStruct((n_seg, sc_cols), jnp.float32),
        scratch_types=(
            [pltpu.VMEM((chunk_rows, w_cols), jnp.float32)] * _NBUF
            + [
                pltpu.VMEM((_LANES,), jnp.float32),
                pltpu.VMEM((w_cols,), jnp.float32),
            ]
            + [pltpu.SemaphoreType.DMA] * _NBUF
        ),
    )
    def sc_pool(x_hbm, sz_hbm, out_hbm, *refs):
        bufs = refs[:_NBUF]
        szv, outv = refs[_NBUF:_NBUF + 2]
        sems = refs[_NBUF + 2:]
        core = lax.axis_index("c")
        sub = lax.axis_index("s")
        seg = core * seg_per_core + sub

        @pl.when(sub < seg_per_core)
        def _():
            row0 = seg * seg_rows

            def src_slice(ci):
                off = pl.multiple_of(row0 + ci * chunk_rows, 8)
                return x_hbm.at[pl.ds(off, chunk_rows), pl.ds(col0, w_cols)]

            for b in range(_NBUF):
                pltpu.async_copy(src_slice(b), bufs[b], sems[b])
            acc0 = tuple(
                jnp.zeros((_LANES,), jnp.float32) for _ in range(n_grp))

            def outer(g, acc):
                for b in range(_NBUF):
                    ci = g * _NBUF + b
                    pltpu.make_async_copy(
                        src_slice(ci), bufs[b], sems[b]).wait()

                    def body(r, carry, b=b):
                        return tuple(
                            carry[gr] + bufs[b][r, pl.ds(gr * _LANES, _LANES)]
                            for gr in range(n_grp)
                        )

                    acc = lax.fori_loop(0, chunk_rows, body, acc)

                    @pl.when(ci + _NBUF < n_chunks)
                    def _(b=b, ci=ci):
                        pltpu.async_copy(
                            src_slice(ci + _NBUF), bufs[b], sems[b])
                return acc

            acc = lax.fori_loop(0, n_chunks // _NBUF, outer, acc0)

            pltpu.sync_copy(sz_hbm.at[seg], szv)
            s = szv[...]
            for g in range(n_grp):
                outv[pl.ds(g * _LANES, _LANES)] = acc[g] / s
            pltpu.sync_copy(outv, out_hbm.at[seg])

    return sc_pool


@functools.lru_cache(maxsize=None)
def _make_tc_pool(n_seg: int, seg_rows: int, units: int, tc_cols: int,
                  blk_rows: int):
    n_blk = seg_rows // blk_rows

    def body(x_ref, r_ref, o_ref):
        @pl.when(pl.program_id(0) == 0)
        def _():
            o_ref[...] = jnp.zeros_like(o_ref)
        o_ref[...] += jnp.sum(x_ref[...], axis=1)

        @pl.when(pl.program_id(0) == n_blk - 1)
        def _():
            o_ref[...] = o_ref[...] / r_ref[...]

    return pl.pallas_call(
        body,
        grid=(n_blk,),
        in_specs=[
            pl.BlockSpec((n_seg, blk_rows, tc_cols), lambda k: (0, k, 0)),
            pl.BlockSpec((n_seg, 1), lambda k: (0, 0)),
        ],
        out_specs=pl.BlockSpec((n_seg, tc_cols), lambda k: (0, 0)),
        out_shape=jax.ShapeDtypeStruct((n_seg, tc_cols), jnp.float32),
    )


def kernel(x, nclasses, nfeature):
    n_seg = nclasses.shape[0]
    units = x.shape[1]
    seg_rows = x.shape[0] // n_seg
    tc_cols = 384
    chunk_rows = 400
    blk_rows = 400

    sizes = (nclasses * nfeature).astype(jnp.float32)
    sz_b = jnp.broadcast_to(sizes[:, None], (n_seg, _LANES))
    x3 = jnp.reshape(x, (n_seg, seg_rows, units))

    sc_out = _make_sc_pool(n_seg, seg_rows, units, tc_cols, chunk_rows)(x, sz_b)
    tc_out = _make_tc_pool(n_seg, seg_rows, units, tc_cols, blk_rows)(
        x3, sizes[:, None])
    return jnp.concatenate([tc_out, sc_out], axis=1)


# speedup vs baseline: 1.0702x; 1.0383x over previous
import functools

import jax
import jax.numpy as jnp
from jax import lax
from jax.experimental import pallas as pl
from jax.experimental.pallas import tpu as pltpu
from jax.experimental.pallas import tpu_sc as plsc

_LANES = 16
_NBUF = 2


@functools.lru_cache(maxsize=None)
def _make_sc_pool(n_seg: int, seg_rows: int, units: int, col0: int,
                  chunk_rows: int):
    n_cores = 1
    sc_cols = units - col0
    w_cols = sc_cols
    seg_per_core = n_seg // n_cores
    n_grp = w_cols // _LANES
    n_chunks = seg_rows // chunk_rows
    assert n_chunks % _NBUF == 0 and chunk_rows % 8 == 0
    mesh = plsc.VectorSubcoreMesh(core_axis_name="c", subcore_axis_name="s",
                                  num_cores=n_cores)

    @functools.partial(
        pl.kernel,
        mesh=mesh,
        out_type=jax.ShapeDtypeStruct((n_seg, sc_cols), jnp.float32),
        scratch_types=(
            [pltpu.VMEM((chunk_rows, w_cols), jnp.float32)] * _NBUF
            + [
                pltpu.VMEM((_LANES,), jnp.float32),
                pltpu.VMEM((w_cols,), jnp.float32),
            ]
            + [pltpu.SemaphoreType.DMA] * _NBUF
        ),
    )
    def sc_pool(x_hbm, sz_hbm, out_hbm, *refs):
        bufs = refs[:_NBUF]
        szv, outv = refs[_NBUF:_NBUF + 2]
        sems = refs[_NBUF + 2:]
        core = lax.axis_index("c")
        sub = lax.axis_index("s")
        seg = core * seg_per_core + sub

        @pl.when(sub < seg_per_core)
        def _():
            row0 = seg * seg_rows

            def src_slice(ci):
                off = pl.multiple_of(row0 + ci * chunk_rows, 8)
                return x_hbm.at[pl.ds(off, chunk_rows), pl.ds(col0, w_cols)]

            for b in range(_NBUF):
                pltpu.async_copy(src_slice(b), bufs[b], sems[b])
            acc0 = tuple(
                jnp.zeros((_LANES,), jnp.float32) for _ in range(n_grp))

            def outer(g, acc):
                for b in range(_NBUF):
                    ci = g * _NBUF + b
                    pltpu.make_async_copy(
                        src_slice(ci), bufs[b], sems[b]).wait()

                    def body(r, carry, b=b):
                        return tuple(
                            carry[gr] + bufs[b][r, pl.ds(gr * _LANES, _LANES)]
                            for gr in range(n_grp)
                        )

                    acc = lax.fori_loop(0, chunk_rows, body, acc)

                    @pl.when(ci + _NBUF < n_chunks)
                    def _(b=b, ci=ci):
                        pltpu.async_copy(
                            src_slice(ci + _NBUF), bufs[b], sems[b])
                return acc

            acc = lax.fori_loop(0, n_chunks // _NBUF, outer, acc0)

            pltpu.sync_copy(sz_hbm.at[seg], szv)
            s = szv[...]
            for g in range(n_grp):
                outv[pl.ds(g * _LANES, _LANES)] = acc[g] / s
            pltpu.sync_copy(outv, out_hbm.at[seg])

    return sc_pool


@functools.lru_cache(maxsize=None)
def _make_tc_pool(n_seg: int, seg_rows: int, units: int, tc_cols: int,
                  blk_rows: int):
    n_blk = seg_rows // blk_rows

    def body(x_ref, r_ref, o_ref):
        @pl.when(pl.program_id(0) == 0)
        def _():
            o_ref[...] = jnp.zeros_like(o_ref)
        o_ref[...] += jnp.sum(x_ref[...], axis=1)

        @pl.when(pl.program_id(0) == n_blk - 1)
        def _():
            o_ref[...] = o_ref[...] / r_ref[...]

    return pl.pallas_call(
        body,
        grid=(n_blk,),
        in_specs=[
            pl.BlockSpec((n_seg, blk_rows, tc_cols), lambda k: (0, k, 0)),
            pl.BlockSpec((n_seg, 1), lambda k: (0, 0)),
        ],
        out_specs=pl.BlockSpec((n_seg, tc_cols), lambda k: (0, 0)),
        out_shape=jax.ShapeDtypeStruct((n_seg, tc_cols), jnp.float32),
    )


def kernel(x, nclasses, nfeature):
    n_seg = nclasses.shape[0]
    units = x.shape[1]
    seg_rows = x.shape[0] // n_seg
    tc_cols = 384
    chunk_rows = 400
    blk_rows = 800

    sizes = (nclasses * nfeature).astype(jnp.float32)
    sz_b = jnp.broadcast_to(sizes[:, None], (n_seg, _LANES))
    x3 = jnp.reshape(x, (n_seg, seg_rows, units))

    sc_out = _make_sc_pool(n_seg, seg_rows, units, tc_cols, chunk_rows)(x, sz_b)
    tc_out = _make_tc_pool(n_seg, seg_rows, units, tc_cols, blk_rows)(
        x3, sizes[:, None])
    return jnp.concatenate([tc_out, sc_out], axis=1)
